# baseline (speedup 1.0000x reference)
import functools

import jax
import jax.numpy as jnp
from jax import lax
from jax.experimental import pallas as pl
from jax.experimental.pallas import tpu as pltpu
from jax.experimental.pallas import tpu_sc as plsc

_INV_STD = 1.0 / (1.0 + 1e-5) ** 0.5

_NW = 32
_CHUNK = 128
_NBUF = 8
_GDEPTH = 4
_RB = 512
_C0F = 0.75


def _make_deg_kernel(n_pad, e_pad):
    per_w = e_pad // _NW
    n_chunks = per_w // _CHUNK
    stripe = n_pad // 16
    mesh = plsc.VectorSubcoreMesh(core_axis_name="c", subcore_axis_name="s")

    @functools.partial(
        pl.kernel,
        mesh=mesh,
        compiler_params=pltpu.CompilerParams(use_tc_tiling_on_sc=False),
        out_type=jax.ShapeDtypeStruct((2, n_pad, 16), jnp.float32),
        scratch_types=[
            pltpu.VMEM((2, 2, _CHUNK), jnp.int32),
            pltpu.VMEM((_CHUNK, 16), jnp.float32),
            pltpu.VMEM_SHARED((n_pad, 16), jnp.float32),
            pltpu.SemaphoreType.DMA,
        ],
    )
    def deg_kernel(es_hbm, z_hbm, ones_hbm, out_hbm, sd, ones_v, acc, ssem):
        cid = lax.axis_index("c")
        sid = lax.axis_index("s")
        wid = sid * 2 + cid
        pltpu.sync_copy(ones_hbm, ones_v)
        pltpu.sync_copy(
            z_hbm.at[pl.ds(sid * stripe, stripe)],
            acc.at[pl.ds(sid * stripe, stripe)],
        )
        plsc.subcore_barrier()
        cbase = wid * n_chunks

        def copy_idx(c, slot):
            pltpu.sync_copy(es_hbm.at[cbase + c], sd.at[slot])

        def scat_start(slot):
            pltpu.async_copy(ones_v, acc.at[sd.at[slot, 1]], ssem, add=True)

        def scat_wait(slot):
            pltpu.make_async_copy(ones_v, acc.at[sd.at[slot, 1]], ssem).wait()

        copy_idx(0, 0)
        scat_start(0)
        copy_idx(1, 1)
        scat_start(1)

        def body(j, carry):
            b = j % 2
            scat_wait(b)
            copy_idx(j, b)
            scat_start(b)
            return carry

        lax.fori_loop(2, n_chunks, body, 0)
        scat_wait(0)
        scat_wait(1)
        plsc.subcore_barrier()
        pltpu.sync_copy(
            acc.at[pl.ds(sid * stripe, stripe)],
            out_hbm.at[cid, pl.ds(sid * stripe, stripe)],
        )

    return deg_kernel


def _make_agg_kernel(n_pad, e_pad, d_feat, n_slabs, c0_frac=0.5):
    per16 = e_pad // _CHUNK // 16
    cb0 = max(_NBUF, min(per16 - _NBUF, round(per16 * c0_frac)))
    cb1 = per16 - cb0
    stripe = n_pad // 16
    mesh = plsc.VectorSubcoreMesh(core_axis_name="c", subcore_axis_name="s")

    def agg_body(*refs):
        hps = refs[:n_slabs]
        es_hbm, z_hbm, out_hbm, sd, rows, acc, gsem, ssem = refs[n_slabs:]
        cid = lax.axis_index("c")
        sid = lax.axis_index("s")
        n_chunks = jnp.where(cid == 0, cb0, cb1)
        cbase = jnp.where(cid == 0, sid * cb0, 16 * cb0 + sid * cb1)

        def copy_idx(c):
            pltpu.sync_copy(es_hbm.at[cbase + c], sd.at[c % _NBUF])

        for slab in range(n_slabs):
            hp_hbm = hps[slab]

            def gather_start(c):
                pltpu.async_copy(
                    hp_hbm.at[sd.at[c % _NBUF, 0]], rows.at[c % _NBUF], gsem)

            def gather_wait(c):
                pltpu.make_async_copy(
                    hp_hbm.at[sd.at[c % _NBUF, 0]], rows.at[c % _NBUF],
                    gsem).wait()

            def scat_start(c):
                pltpu.async_copy(
                    rows.at[c % _NBUF], acc.at[sd.at[c % _NBUF, 1]], ssem,
                    add=True)

            def scat_wait(c):
                pltpu.make_async_copy(
                    rows.at[c % _NBUF], acc.at[sd.at[c % _NBUF, 1]],
                    ssem).wait()

            pltpu.sync_copy(
                z_hbm.at[pl.ds(sid * stripe, stripe)],
                acc.at[pl.ds(sid * stripe, stripe)],
            )
            plsc.subcore_barrier()

            for c in range(_GDEPTH):
                copy_idx(c)
                gather_start(c)

            def body(j, carry):
                gather_wait(j)
                scat_start(j)

                @pl.when(jnp.logical_and(j + _GDEPTH < n_chunks,
                                         j >= _GDEPTH))
                def _():
                    scat_wait(j - (_NBUF - _GDEPTH))

                @pl.when(j + _GDEPTH < n_chunks)
                def _():
                    copy_idx(j + _GDEPTH)
                    gather_start(j + _GDEPTH)

                return carry

            lax.fori_loop(0, n_chunks, body, 0)
            for c in range(_NBUF):
                scat_wait(c)
            plsc.subcore_barrier()
            pltpu.sync_copy(
                acc.at[pl.ds(sid * stripe, stripe)],
                out_hbm.at[slab, cid, pl.ds(sid * stripe, stripe)],
            )

    return functools.partial(
        pl.kernel,
        mesh=mesh,
        compiler_params=pltpu.CompilerParams(use_tc_tiling_on_sc=False),
        out_type=jax.ShapeDtypeStruct((n_slabs, 2, n_pad, d_feat),
                                      jnp.float32),
        scratch_types=[
            pltpu.VMEM((_NBUF, 2, _CHUNK), jnp.int32),
            pltpu.VMEM((_NBUF, _CHUNK, d_feat), jnp.float32),
            pltpu.VMEM_SHARED((n_pad, d_feat), jnp.float32),
            pltpu.SemaphoreType.DMA,
            pltpu.SemaphoreType.DMA,
        ],
    )(agg_body)


def _full(shape):
    nd = len(shape)
    return pl.BlockSpec(shape, lambda i, _n=nd: (0,) * _n)


def _rows(minor):
    return pl.BlockSpec((_RB, minor), lambda i: (i, 0))


def _deg_part(which):
    return pl.BlockSpec((1, _RB, 16), lambda i, _w=which: (_w, i, 0))


def _agg_part(minor, slab, which):
    return pl.BlockSpec((1, 1, _RB, minor),
                        lambda i, _s=slab, _w=which: (_s, _w, i, 0))


def _stage_a(n_pad):
    def body(d0_ref, d1_ref, x_ref, w1_ref, dinv_ref, hpa_ref, hpb_ref):
        deg = d0_ref[0] + d1_ref[0] + 1.0
        dv = lax.rsqrt(deg)
        dinv_ref[...] = dv
        xw = jnp.dot(x_ref[...], w1_ref[...], preferred_element_type=jnp.float32)
        hp = xw * dv[:, 0:1]
        hpa_ref[...] = hp[:, :64]
        hpb_ref[...] = hp[:, 64:]

    return pl.pallas_call(
        body,
        grid=(n_pad // _RB,),
        in_specs=[_deg_part(0), _deg_part(1), _rows(128), _full((128, 128))],
        out_specs=[_rows(16), _rows(64), _rows(64)],
        out_shape=[
            jax.ShapeDtypeStruct((n_pad, 16), jnp.float32),
            jax.ShapeDtypeStruct((n_pad, 64), jnp.float32),
            jax.ShapeDtypeStruct((n_pad, 64), jnp.float32),
        ],
    )


def _stage_b1(n_pad):
    def body(aa0_ref, aa1_ref, ab0_ref, ab1_ref, hpa_ref, hpb_ref, x_ref,
             dv_ref, b_ref, g_ref, be_ref, wn_ref, swn_ref, sbn_ref,
             hpn_ref, resn_ref):
        dv = dv_ref[...][:, 0:1]
        ta = aa0_ref[0, 0] + aa1_ref[0, 0] + hpa_ref[...]
        tb = ab0_ref[0, 0] + ab1_ref[0, 0] + hpb_ref[...]
        t = jnp.concatenate([ta, tb], axis=1) * dv + b_ref[...][None, :]
        t = t * (_INV_STD * g_ref[...][None, :]) + be_ref[...][None, :]
        h = jnp.maximum(t, 0.0) + x_ref[...]
        hpn_ref[...] = jnp.dot(h, wn_ref[...],
                               preferred_element_type=jnp.float32) * dv
        resn_ref[...] = jnp.dot(h, swn_ref[...],
                                preferred_element_type=jnp.float32) + sbn_ref[...][None, :]

    return pl.pallas_call(
        body,
        grid=(n_pad // _RB,),
        in_specs=[
            _agg_part(64, 0, 0), _agg_part(64, 0, 1),
            _agg_part(64, 1, 0), _agg_part(64, 1, 1),
            _rows(64), _rows(64), _rows(128),
            _rows(16), _full((128,)), _full((128,)), _full((128,)),
            _full((128, 64)), _full((128, 64)), _full((64,)),
        ],
        out_specs=[_rows(64), _rows(64)],
        out_shape=[
            jax.ShapeDtypeStruct((n_pad, 64), jnp.float32),
            jax.ShapeDtypeStruct((n_pad, 64), jnp.float32),
        ],
    )


def _stage_b2(n_pad):
    def body(a0_ref, a1_ref, hp_ref, res_ref, dv_ref, b_ref, g_ref, be_ref,
             wn_ref, swn_ref, sbn_ref, hpn_ref, resn_ref):
        dv = dv_ref[...][:, 0:1]
        t = (a0_ref[0, 0] + a1_ref[0, 0] + hp_ref[...]) * dv + b_ref[...][None, :]
        t = t * (_INV_STD * g_ref[...][None, :]) + be_ref[...][None, :]
        h = jnp.maximum(t, 0.0) + res_ref[...]
        hpn_ref[...] = jnp.dot(h, wn_ref[...],
                               preferred_element_type=jnp.float32) * dv
        resn_ref[...] = jnp.dot(h, swn_ref[...],
                                preferred_element_type=jnp.float32) + sbn_ref[...][None, :]

    return pl.pallas_call(
        body,
        grid=(n_pad // _RB,),
        in_specs=[
            _agg_part(64, 0, 0), _agg_part(64, 0, 1),
            _rows(64), _rows(64),
            _rows(16), _full((64,)), _full((64,)), _full((64,)),
            _full((64, 32)), _full((64, 32)), _full((32,)),
        ],
        out_specs=[_rows(32), _rows(32)],
        out_shape=[
            jax.ShapeDtypeStruct((n_pad, 32), jnp.float32),
            jax.ShapeDtypeStruct((n_pad, 32), jnp.float32),
        ],
    )


def _stage_final(n_pad):
    def body(a0_ref, a1_ref, hp_ref, res_ref, dv_ref, b_ref, g_ref, be_ref,
             cw1_ref, cb1_ref, cw2_ref, cb2_ref, out_ref):
        dv = dv_ref[...][:, 0:1]
        t = (a0_ref[0, 0] + a1_ref[0, 0] + hp_ref[...]) * dv + b_ref[...][None, :]
        t = t * (_INV_STD * g_ref[...][None, :]) + be_ref[...][None, :]
        h = jnp.maximum(t, 0.0) + res_ref[...]
        u = jnp.dot(h, cw1_ref[...], preferred_element_type=jnp.float32)
        u = jnp.maximum(u + cb1_ref[...][None, :], 0.0)
        v = jnp.dot(u, cw2_ref[...],
                    preferred_element_type=jnp.float32) + cb2_ref[...][None, :]
        m = jnp.max(v, axis=1, keepdims=True)
        lse = m + jnp.log(jnp.sum(jnp.exp(v - m), axis=1, keepdims=True))
        out_ref[...] = v - lse

    return pl.pallas_call(
        body,
        grid=(n_pad // _RB,),
        in_specs=[
            _agg_part(32, 0, 0), _agg_part(32, 0, 1),
            _rows(32), _rows(32),
            _rows(16), _full((32,)), _full((32,)), _full((32,)),
            _full((32, 16)), _full((16,)), _full((16, 2)), _full((2,)),
        ],
        out_specs=_rows(2),
        out_shape=jax.ShapeDtypeStruct((n_pad, 2), jnp.float32),
    )


def kernel(x, edge_index, W1, b1, g1, be1, W2, b2, g2, be2, W3, b3, g3, be3,
           sW2, sb2, sW3, sb3, cW1, cb1, cW2, cb2):
    n, _ = x.shape
    e = edge_index.shape[1]
    n_pad = -(-n // 2048) * 2048
    e_pad = -(-e // (_NW * _CHUNK)) * (_NW * _CHUNK)

    pad = e_pad - e
    s_pad = jnp.concatenate([edge_index[0], jnp.zeros((pad,), jnp.int32)])
    d_pad = jnp.concatenate(
        [edge_index[1], jnp.full((pad,), n_pad - 1, jnp.int32)])
    es = jnp.stack([s_pad.reshape(-1, _CHUNK), d_pad.reshape(-1, _CHUNK)],
                   axis=1)
    x_p = jnp.pad(x, ((0, n_pad - n), (0, 0)))

    z64 = jnp.zeros((n_pad, 64), jnp.float32)
    ones16 = jnp.ones((_CHUNK, 16), jnp.float32)

    deg_p = _make_deg_kernel(n_pad, e_pad)(es, z64[:, :16], ones16)
    dinv, hp1a, hp1b = _stage_a(n_pad)(deg_p, deg_p, x_p, W1)

    agg1 = _make_agg_kernel(n_pad, e_pad, 64, 2, _C0F)(hp1a, hp1b, es, z64)
    hp2, res2 = _stage_b1(n_pad)(
        agg1, agg1, agg1, agg1, hp1a, hp1b, x_p, dinv,
        b1, g1, be1, W2, sW2, sb2)

    agg2 = _make_agg_kernel(n_pad, e_pad, 64, 1, _C0F)(hp2, es, z64)
    hp3, res3 = _stage_b2(n_pad)(
        agg2, agg2, hp2, res2, dinv, b2, g2, be2, W3, sW3, sb3)

    agg3 = _make_agg_kernel(n_pad, e_pad, 32, 1, _C0F)(hp3, es, z64[:, :32])
    out = _stage_final(n_pad)(
        agg3, agg3, hp3, res3, dinv, b3, g3, be3, cW1, cb1, cW2, cb2)

    return out[:n]


# speedup vs baseline: 1.0098x; 1.0098x over previous
import functools

import jax
import jax.numpy as jnp
from jax import lax
from jax.experimental import pallas as pl
from jax.experimental.pallas import tpu as pltpu
from jax.experimental.pallas import tpu_sc as plsc

_INV_STD = 1.0 / (1.0 + 1e-5) ** 0.5

_NW = 32
_CHUNK = 128
_NBUF = 6
_GDEPTH = 3
_RB = 512
_C0F = 0.75


def _make_deg_kernel(n_pad, e_pad):
    per_w = e_pad // _NW
    n_chunks = per_w // _CHUNK
    stripe = n_pad // 16
    mesh = plsc.VectorSubcoreMesh(core_axis_name="c", subcore_axis_name="s")

    @functools.partial(
        pl.kernel,
        mesh=mesh,
        compiler_params=pltpu.CompilerParams(use_tc_tiling_on_sc=False),
        out_type=jax.ShapeDtypeStruct((2, n_pad, 16), jnp.float32),
        scratch_types=[
            pltpu.VMEM((2, 2, _CHUNK), jnp.int32),
            pltpu.VMEM((_CHUNK, 16), jnp.float32),
            pltpu.VMEM_SHARED((n_pad, 16), jnp.float32),
            pltpu.SemaphoreType.DMA,
        ],
    )
    def deg_kernel(es_hbm, z_hbm, ones_hbm, out_hbm, sd, ones_v, acc, ssem):
        cid = lax.axis_index("c")
        sid = lax.axis_index("s")
        wid = sid * 2 + cid
        pltpu.sync_copy(ones_hbm, ones_v)
        pltpu.sync_copy(
            z_hbm.at[pl.ds(sid * stripe, stripe)],
            acc.at[pl.ds(sid * stripe, stripe)],
        )
        plsc.subcore_barrier()
        cbase = wid * n_chunks

        def copy_idx(c, slot):
            pltpu.sync_copy(es_hbm.at[cbase + c], sd.at[slot])

        def scat_start(slot):
            pltpu.async_copy(ones_v, acc.at[sd.at[slot, 1]], ssem, add=True)

        def scat_wait(slot):
            pltpu.make_async_copy(ones_v, acc.at[sd.at[slot, 1]], ssem).wait()

        copy_idx(0, 0)
        scat_start(0)
        copy_idx(1, 1)
        scat_start(1)

        def body(j, carry):
            b = j % 2
            scat_wait(b)
            copy_idx(j, b)
            scat_start(b)
            return carry

        lax.fori_loop(2, n_chunks, body, 0)
        scat_wait(0)
        scat_wait(1)
        plsc.subcore_barrier()
        pltpu.sync_copy(
            acc.at[pl.ds(sid * stripe, stripe)],
            out_hbm.at[cid, pl.ds(sid * stripe, stripe)],
        )

    return deg_kernel


def _make_agg_kernel(n_pad, e_pad, d_feat, n_slabs, c0_frac=0.5):
    per16 = e_pad // _CHUNK // 16
    cb0 = max(_NBUF, min(per16 - _NBUF, round(per16 * c0_frac)))
    cb1 = per16 - cb0
    stripe = n_pad // 16
    mesh = plsc.VectorSubcoreMesh(core_axis_name="c", subcore_axis_name="s")

    def agg_body(*refs):
        hps = refs[:n_slabs]
        es_hbm, z_hbm, out_hbm, sd, rows, acc, gsem, ssem = refs[n_slabs:]
        cid = lax.axis_index("c")
        sid = lax.axis_index("s")
        n_chunks = jnp.where(cid == 0, cb0, cb1)
        cbase = jnp.where(cid == 0, sid * cb0, 16 * cb0 + sid * cb1)

        def copy_idx(c):
            pltpu.sync_copy(es_hbm.at[cbase + c], sd.at[c % _NBUF])

        for slab in range(n_slabs):
            hp_hbm = hps[slab]

            def gather_start(c):
                pltpu.async_copy(
                    hp_hbm.at[sd.at[c % _NBUF, 0]], rows.at[c % _NBUF], gsem)

            def gather_wait(c):
                pltpu.make_async_copy(
                    hp_hbm.at[sd.at[c % _NBUF, 0]], rows.at[c % _NBUF],
                    gsem).wait()

            def scat_start(c):
                pltpu.async_copy(
                    rows.at[c % _NBUF], acc.at[sd.at[c % _NBUF, 1]], ssem,
                    add=True)

            def scat_wait(c):
                pltpu.make_async_copy(
                    rows.at[c % _NBUF], acc.at[sd.at[c % _NBUF, 1]],
                    ssem).wait()

            for c in range(_GDEPTH):
                copy_idx(c)
                gather_start(c)

            pltpu.sync_copy(
                z_hbm.at[pl.ds(sid * stripe, stripe)],
                acc.at[pl.ds(sid * stripe, stripe)],
            )
            plsc.subcore_barrier()

            def body(j, carry):
                gather_wait(j)
                scat_start(j)

                @pl.when(jnp.logical_and(j + _GDEPTH < n_chunks,
                                         j >= _GDEPTH))
                def _():
                    scat_wait(j - (_NBUF - _GDEPTH))

                @pl.when(j + _GDEPTH < n_chunks)
                def _():
                    copy_idx(j + _GDEPTH)
                    gather_start(j + _GDEPTH)

                return carry

            lax.fori_loop(0, n_chunks, body, 0)
            for c in range(_NBUF):
                scat_wait(c)
            plsc.subcore_barrier()
            pltpu.sync_copy(
                acc.at[pl.ds(sid * stripe, stripe)],
                out_hbm.at[slab, cid, pl.ds(sid * stripe, stripe)],
            )

    return functools.partial(
        pl.kernel,
        mesh=mesh,
        compiler_params=pltpu.CompilerParams(use_tc_tiling_on_sc=False),
        out_type=jax.ShapeDtypeStruct((n_slabs, 2, n_pad, d_feat),
                                      jnp.float32),
        scratch_types=[
            pltpu.VMEM((_NBUF, 2, _CHUNK), jnp.int32),
            pltpu.VMEM((_NBUF, _CHUNK, d_feat), jnp.float32),
            pltpu.VMEM_SHARED((n_pad, d_feat), jnp.float32),
            pltpu.SemaphoreType.DMA,
            pltpu.SemaphoreType.DMA,
        ],
    )(agg_body)


def _full(shape):
    nd = len(shape)
    return pl.BlockSpec(shape, lambda i, _n=nd: (0,) * _n)


def _rows(minor):
    return pl.BlockSpec((_RB, minor), lambda i: (i, 0))


def _deg_part(which):
    return pl.BlockSpec((1, _RB, 16), lambda i, _w=which: (_w, i, 0))


def _agg_part(minor, slab, which):
    return pl.BlockSpec((1, 1, _RB, minor),
                        lambda i, _s=slab, _w=which: (_s, _w, i, 0))


def _stage_a(n_pad):
    def body(d0_ref, d1_ref, x_ref, w1_ref, dinv_ref, hpa_ref, hpb_ref):
        deg = d0_ref[0] + d1_ref[0] + 1.0
        dv = lax.rsqrt(deg)
        dinv_ref[...] = dv
        xw = jnp.dot(x_ref[...], w1_ref[...], preferred_element_type=jnp.float32)
        hp = xw * dv[:, 0:1]
        hpa_ref[...] = hp[:, :64]
        hpb_ref[...] = hp[:, 64:]

    return pl.pallas_call(
        body,
        grid=(n_pad // _RB,),
        in_specs=[_deg_part(0), _deg_part(1), _rows(128), _full((128, 128))],
        out_specs=[_rows(16), _rows(64), _rows(64)],
        out_shape=[
            jax.ShapeDtypeStruct((n_pad, 16), jnp.float32),
            jax.ShapeDtypeStruct((n_pad, 64), jnp.float32),
            jax.ShapeDtypeStruct((n_pad, 64), jnp.float32),
        ],
    )


def _stage_b1(n_pad):
    def body(aa0_ref, aa1_ref, ab0_ref, ab1_ref, hpa_ref, hpb_ref, x_ref,
             dv_ref, b_ref, g_ref, be_ref, wn_ref, swn_ref, sbn_ref,
             hpn_ref, resn_ref):
        dv = dv_ref[...][:, 0:1]
        ta = aa0_ref[0, 0] + aa1_ref[0, 0] + hpa_ref[...]
        tb = ab0_ref[0, 0] + ab1_ref[0, 0] + hpb_ref[...]
        t = jnp.concatenate([ta, tb], axis=1) * dv + b_ref[...][None, :]
        t = t * (_INV_STD * g_ref[...][None, :]) + be_ref[...][None, :]
        h = jnp.maximum(t, 0.0) + x_ref[...]
        hpn_ref[...] = jnp.dot(h, wn_ref[...],
                               preferred_element_type=jnp.float32) * dv
        resn_ref[...] = jnp.dot(h, swn_ref[...],
                                preferred_element_type=jnp.float32) + sbn_ref[...][None, :]

    return pl.pallas_call(
        body,
        grid=(n_pad // _RB,),
        in_specs=[
            _agg_part(64, 0, 0), _agg_part(64, 0, 1),
            _agg_part(64, 1, 0), _agg_part(64, 1, 1),
            _rows(64), _rows(64), _rows(128),
            _rows(16), _full((128,)), _full((128,)), _full((128,)),
            _full((128, 64)), _full((128, 64)), _full((64,)),
        ],
        out_specs=[_rows(64), _rows(64)],
        out_shape=[
            jax.ShapeDtypeStruct((n_pad, 64), jnp.float32),
            jax.ShapeDtypeStruct((n_pad, 64), jnp.float32),
        ],
    )


def _stage_b2(n_pad):
    def body(a0_ref, a1_ref, hp_ref, res_ref, dv_ref, b_ref, g_ref, be_ref,
             wn_ref, swn_ref, sbn_ref, hpn_ref, resn_ref):
        dv = dv_ref[...][:, 0:1]
        t = (a0_ref[0, 0] + a1_ref[0, 0] + hp_ref[...]) * dv + b_ref[...][None, :]
        t = t * (_INV_STD * g_ref[...][None, :]) + be_ref[...][None, :]
        h = jnp.maximum(t, 0.0) + res_ref[...]
        hpn_ref[...] = jnp.dot(h, wn_ref[...],
                               preferred_element_type=jnp.float32) * dv
        resn_ref[...] = jnp.dot(h, swn_ref[...],
                                preferred_element_type=jnp.float32) + sbn_ref[...][None, :]

    return pl.pallas_call(
        body,
        grid=(n_pad // _RB,),
        in_specs=[
            _agg_part(64, 0, 0), _agg_part(64, 0, 1),
            _rows(64), _rows(64),
            _rows(16), _full((64,)), _full((64,)), _full((64,)),
            _full((64, 32)), _full((64, 32)), _full((32,)),
        ],
        out_specs=[_rows(32), _rows(32)],
        out_shape=[
            jax.ShapeDtypeStruct((n_pad, 32), jnp.float32),
            jax.ShapeDtypeStruct((n_pad, 32), jnp.float32),
        ],
    )


def _stage_final(n_pad):
    def body(a0_ref, a1_ref, hp_ref, res_ref, dv_ref, b_ref, g_ref, be_ref,
             cw1_ref, cb1_ref, cw2_ref, cb2_ref, out_ref):
        dv = dv_ref[...][:, 0:1]
        t = (a0_ref[0, 0] + a1_ref[0, 0] + hp_ref[...]) * dv + b_ref[...][None, :]
        t = t * (_INV_STD * g_ref[...][None, :]) + be_ref[...][None, :]
        h = jnp.maximum(t, 0.0) + res_ref[...]
        u = jnp.dot(h, cw1_ref[...], preferred_element_type=jnp.float32)
        u = jnp.maximum(u + cb1_ref[...][None, :], 0.0)
        v = jnp.dot(u, cw2_ref[...],
                    preferred_element_type=jnp.float32) + cb2_ref[...][None, :]
        m = jnp.max(v, axis=1, keepdims=True)
        lse = m + jnp.log(jnp.sum(jnp.exp(v - m), axis=1, keepdims=True))
        out_ref[...] = v - lse

    return pl.pallas_call(
        body,
        grid=(n_pad // _RB,),
        in_specs=[
            _agg_part(32, 0, 0), _agg_part(32, 0, 1),
            _rows(32), _rows(32),
            _rows(16), _full((32,)), _full((32,)), _full((32,)),
            _full((32, 16)), _full((16,)), _full((16, 2)), _full((2,)),
        ],
        out_specs=_rows(2),
        out_shape=jax.ShapeDtypeStruct((n_pad, 2), jnp.float32),
    )


def kernel(x, edge_index, W1, b1, g1, be1, W2, b2, g2, be2, W3, b3, g3, be3,
           sW2, sb2, sW3, sb3, cW1, cb1, cW2, cb2):
    n, _ = x.shape
    e = edge_index.shape[1]
    n_pad = -(-n // 2048) * 2048
    e_pad = -(-e // (_NW * _CHUNK)) * (_NW * _CHUNK)

    pad = e_pad - e
    s_pad = jnp.concatenate([edge_index[0], jnp.zeros((pad,), jnp.int32)])
    d_pad = jnp.concatenate(
        [edge_index[1], jnp.full((pad,), n_pad - 1, jnp.int32)])
    es = jnp.stack([s_pad.reshape(-1, _CHUNK), d_pad.reshape(-1, _CHUNK)],
                   axis=1)
    x_p = jnp.pad(x, ((0, n_pad - n), (0, 0)))

    z64 = jnp.zeros((n_pad, 64), jnp.float32)
    ones16 = jnp.ones((_CHUNK, 16), jnp.float32)

    deg_p = _make_deg_kernel(n_pad, e_pad)(es, z64[:, :16], ones16)
    dinv, hp1a, hp1b = _stage_a(n_pad)(deg_p, deg_p, x_p, W1)

    agg1 = _make_agg_kernel(n_pad, e_pad, 64, 2, _C0F)(hp1a, hp1b, es, z64)
    hp2, res2 = _stage_b1(n_pad)(
        agg1, agg1, agg1, agg1, hp1a, hp1b, x_p, dinv,
        b1, g1, be1, W2, sW2, sb2)

    agg2 = _make_agg_kernel(n_pad, e_pad, 64, 1, _C0F)(hp2, es, z64)
    hp3, res3 = _stage_b2(n_pad)(
        agg2, agg2, hp2, res2, dinv, b2, g2, be2, W3, sW3, sb3)

    agg3 = _make_agg_kernel(n_pad, e_pad, 32, 1, _C0F)(hp3, es, z64[:, :32])
    out = _stage_final(n_pad)(
        agg3, agg3, hp3, res3, dinv, b3, g3, be3, cW1, cb1, cW2, cb2)

    return out[:n]


# speedup vs baseline: 1.0250x; 1.0150x over previous
import functools

import jax
import jax.numpy as jnp
from jax import lax
from jax.experimental import pallas as pl
from jax.experimental.pallas import tpu as pltpu
from jax.experimental.pallas import tpu_sc as plsc

_INV_STD = 1.0 / (1.0 + 1e-5) ** 0.5

_NW = 32
_CHUNK = 128
_NBUF = 6
_GDEPTH = 3
_RB = 512
_C0F = 0.8


def _make_deg_kernel(n_pad, e_pad):
    per_w = e_pad // _NW
    n_chunks = per_w // _CHUNK
    stripe = n_pad // 16
    mesh = plsc.VectorSubcoreMesh(core_axis_name="c", subcore_axis_name="s")

    @functools.partial(
        pl.kernel,
        mesh=mesh,
        compiler_params=pltpu.CompilerParams(use_tc_tiling_on_sc=False),
        out_type=jax.ShapeDtypeStruct((2, n_pad, 16), jnp.float32),
        scratch_types=[
            pltpu.VMEM((2, 2, _CHUNK), jnp.int32),
            pltpu.VMEM((_CHUNK, 16), jnp.float32),
            pltpu.VMEM_SHARED((n_pad, 16), jnp.float32),
            pltpu.SemaphoreType.DMA,
        ],
    )
    def deg_kernel(es_hbm, z_hbm, ones_hbm, out_hbm, sd, ones_v, acc, ssem):
        cid = lax.axis_index("c")
        sid = lax.axis_index("s")
        wid = sid * 2 + cid
        pltpu.sync_copy(ones_hbm, ones_v)
        pltpu.sync_copy(
            z_hbm.at[pl.ds(sid * stripe, stripe)],
            acc.at[pl.ds(sid * stripe, stripe)],
        )
        plsc.subcore_barrier()
        cbase = wid * n_chunks

        def copy_idx(c, slot):
            pltpu.sync_copy(es_hbm.at[cbase + c], sd.at[slot])

        def scat_start(slot):
            pltpu.async_copy(ones_v, acc.at[sd.at[slot, 1]], ssem, add=True)

        def scat_wait(slot):
            pltpu.make_async_copy(ones_v, acc.at[sd.at[slot, 1]], ssem).wait()

        copy_idx(0, 0)
        scat_start(0)
        copy_idx(1, 1)
        scat_start(1)

        def body(j, carry):
            b = j % 2
            scat_wait(b)
            copy_idx(j, b)
            scat_start(b)
            return carry

        lax.fori_loop(2, n_chunks, body, 0)
        scat_wait(0)
        scat_wait(1)
        plsc.subcore_barrier()
        pltpu.sync_copy(
            acc.at[pl.ds(sid * stripe, stripe)],
            out_hbm.at[cid, pl.ds(sid * stripe, stripe)],
        )

    return deg_kernel


def _make_agg_kernel(n_pad, e_pad, d_feat, n_slabs, c0_frac=0.5):
    per16 = e_pad // _CHUNK // 16
    cb0 = max(_NBUF, min(per16 - _NBUF, round(per16 * c0_frac)))
    cb1 = per16 - cb0
    stripe = n_pad // 16
    mesh = plsc.VectorSubcoreMesh(core_axis_name="c", subcore_axis_name="s")

    def agg_body(*refs):
        hps = refs[:n_slabs]
        es_hbm, z_hbm, out_hbm, sd, rows, acc, gsem, ssem = refs[n_slabs:]
        cid = lax.axis_index("c")
        sid = lax.axis_index("s")
        n_chunks = jnp.where(cid == 0, cb0, cb1)
        cbase = jnp.where(cid == 0, sid * cb0, 16 * cb0 + sid * cb1)

        def copy_idx(c):
            pltpu.sync_copy(es_hbm.at[cbase + c], sd.at[c % _NBUF])

        for slab in range(n_slabs):
            hp_hbm = hps[slab]

            def gather_start(c):
                pltpu.async_copy(
                    hp_hbm.at[sd.at[c % _NBUF, 0]], rows.at[c % _NBUF], gsem)

            def gather_wait(c):
                pltpu.make_async_copy(
                    hp_hbm.at[sd.at[c % _NBUF, 0]], rows.at[c % _NBUF],
                    gsem).wait()

            def scat_start(c):
                pltpu.async_copy(
                    rows.at[c % _NBUF], acc.at[sd.at[c % _NBUF, 1]], ssem,
                    add=True)

            def scat_wait(c):
                pltpu.make_async_copy(
                    rows.at[c % _NBUF], acc.at[sd.at[c % _NBUF, 1]],
                    ssem).wait()

            for c in range(_GDEPTH):
                copy_idx(c)
                gather_start(c)

            pltpu.sync_copy(
                z_hbm.at[pl.ds(sid * stripe, stripe)],
                acc.at[pl.ds(sid * stripe, stripe)],
            )
            plsc.subcore_barrier()

            def body(j, carry):
                gather_wait(j)
                scat_start(j)

                @pl.when(jnp.logical_and(j + _GDEPTH < n_chunks,
                                         j >= _GDEPTH))
                def _():
                    scat_wait(j - (_NBUF - _GDEPTH))

                @pl.when(j + _GDEPTH < n_chunks)
                def _():
                    copy_idx(j + _GDEPTH)
                    gather_start(j + _GDEPTH)

                return carry

            lax.fori_loop(0, n_chunks, body, 0)
            for c in range(_NBUF):
                scat_wait(c)
            plsc.subcore_barrier()
            pltpu.sync_copy(
                acc.at[pl.ds(sid * stripe, stripe)],
                out_hbm.at[slab, cid, pl.ds(sid * stripe, stripe)],
            )

    return functools.partial(
        pl.kernel,
        mesh=mesh,
        compiler_params=pltpu.CompilerParams(use_tc_tiling_on_sc=False),
        out_type=jax.ShapeDtypeStruct((n_slabs, 2, n_pad, d_feat),
                                      jnp.float32),
        scratch_types=[
            pltpu.VMEM((_NBUF, 2, _CHUNK), jnp.int32),
            pltpu.VMEM((_NBUF, _CHUNK, d_feat), jnp.float32),
            pltpu.VMEM_SHARED((n_pad, d_feat), jnp.float32),
            pltpu.SemaphoreType.DMA,
            pltpu.SemaphoreType.DMA,
        ],
    )(agg_body)


def _full(shape):
    nd = len(shape)
    return pl.BlockSpec(shape, lambda i, _n=nd: (0,) * _n)


def _rows(minor):
    return pl.BlockSpec((_RB, minor), lambda i: (i, 0))


def _deg_part(which):
    return pl.BlockSpec((1, _RB, 16), lambda i, _w=which: (_w, i, 0))


def _agg_part(minor, slab, which):
    return pl.BlockSpec((1, 1, _RB, minor),
                        lambda i, _s=slab, _w=which: (_s, _w, i, 0))


def _stage_a(n_pad):
    def body(d0_ref, d1_ref, x_ref, w1_ref, dinv_ref, hpa_ref, hpb_ref):
        deg = d0_ref[0] + d1_ref[0] + 1.0
        dv = lax.rsqrt(deg)
        dinv_ref[...] = dv
        xw = jnp.dot(x_ref[...], w1_ref[...], preferred_element_type=jnp.float32)
        hp = xw * dv[:, 0:1]
        hpa_ref[...] = hp[:, :64]
        hpb_ref[...] = hp[:, 64:]

    return pl.pallas_call(
        body,
        grid=(n_pad // _RB,),
        in_specs=[_deg_part(0), _deg_part(1), _rows(128), _full((128, 128))],
        out_specs=[_rows(16), _rows(64), _rows(64)],
        out_shape=[
            jax.ShapeDtypeStruct((n_pad, 16), jnp.float32),
            jax.ShapeDtypeStruct((n_pad, 64), jnp.float32),
            jax.ShapeDtypeStruct((n_pad, 64), jnp.float32),
        ],
    )


def _stage_b1(n_pad):
    def body(aa0_ref, aa1_ref, ab0_ref, ab1_ref, hpa_ref, hpb_ref, x_ref,
             dv_ref, b_ref, g_ref, be_ref, wn_ref, swn_ref, sbn_ref,
             hpn_ref, resn_ref):
        dv = dv_ref[...][:, 0:1]
        ta = aa0_ref[0, 0] + aa1_ref[0, 0] + hpa_ref[...]
        tb = ab0_ref[0, 0] + ab1_ref[0, 0] + hpb_ref[...]
        t = jnp.concatenate([ta, tb], axis=1) * dv + b_ref[...][None, :]
        t = t * (_INV_STD * g_ref[...][None, :]) + be_ref[...][None, :]
        h = jnp.maximum(t, 0.0) + x_ref[...]
        hpn_ref[...] = jnp.dot(h, wn_ref[...],
                               preferred_element_type=jnp.float32) * dv
        resn_ref[...] = jnp.dot(h, swn_ref[...],
                                preferred_element_type=jnp.float32) + sbn_ref[...][None, :]

    return pl.pallas_call(
        body,
        grid=(n_pad // _RB,),
        in_specs=[
            _agg_part(64, 0, 0), _agg_part(64, 0, 1),
            _agg_part(64, 1, 0), _agg_part(64, 1, 1),
            _rows(64), _rows(64), _rows(128),
            _rows(16), _full((128,)), _full((128,)), _full((128,)),
            _full((128, 64)), _full((128, 64)), _full((64,)),
        ],
        out_specs=[_rows(64), _rows(64)],
        out_shape=[
            jax.ShapeDtypeStruct((n_pad, 64), jnp.float32),
            jax.ShapeDtypeStruct((n_pad, 64), jnp.float32),
        ],
    )


def _stage_b2(n_pad):
    def body(a0_ref, a1_ref, hp_ref, res_ref, dv_ref, b_ref, g_ref, be_ref,
             wn_ref, swn_ref, sbn_ref, hpn_ref, resn_ref):
        dv = dv_ref[...][:, 0:1]
        t = (a0_ref[0, 0] + a1_ref[0, 0] + hp_ref[...]) * dv + b_ref[...][None, :]
        t = t * (_INV_STD * g_ref[...][None, :]) + be_ref[...][None, :]
        h = jnp.maximum(t, 0.0) + res_ref[...]
        hpn_ref[...] = jnp.dot(h, wn_ref[...],
                               preferred_element_type=jnp.float32) * dv
        resn_ref[...] = jnp.dot(h, swn_ref[...],
                                preferred_element_type=jnp.float32) + sbn_ref[...][None, :]

    return pl.pallas_call(
        body,
        grid=(n_pad // _RB,),
        in_specs=[
            _agg_part(64, 0, 0), _agg_part(64, 0, 1),
            _rows(64), _rows(64),
            _rows(16), _full((64,)), _full((64,)), _full((64,)),
            _full((64, 32)), _full((64, 32)), _full((32,)),
        ],
        out_specs=[_rows(32), _rows(32)],
        out_shape=[
            jax.ShapeDtypeStruct((n_pad, 32), jnp.float32),
            jax.ShapeDtypeStruct((n_pad, 32), jnp.float32),
        ],
    )


def _stage_final(n_pad):
    def body(a0_ref, a1_ref, hp_ref, res_ref, dv_ref, b_ref, g_ref, be_ref,
             cw1_ref, cb1_ref, cw2_ref, cb2_ref, out_ref):
        dv = dv_ref[...][:, 0:1]
        t = (a0_ref[0, 0] + a1_ref[0, 0] + hp_ref[...]) * dv + b_ref[...][None, :]
        t = t * (_INV_STD * g_ref[...][None, :]) + be_ref[...][None, :]
        h = jnp.maximum(t, 0.0) + res_ref[...]
        u = jnp.dot(h, cw1_ref[...], preferred_element_type=jnp.float32)
        u = jnp.maximum(u + cb1_ref[...][None, :], 0.0)
        v = jnp.dot(u, cw2_ref[...],
                    preferred_element_type=jnp.float32) + cb2_ref[...][None, :]
        m = jnp.max(v, axis=1, keepdims=True)
        lse = m + jnp.log(jnp.sum(jnp.exp(v - m), axis=1, keepdims=True))
        out_ref[...] = v - lse

    return pl.pallas_call(
        body,
        grid=(n_pad // _RB,),
        in_specs=[
            _agg_part(32, 0, 0), _agg_part(32, 0, 1),
            _rows(32), _rows(32),
            _rows(16), _full((32,)), _full((32,)), _full((32,)),
            _full((32, 16)), _full((16,)), _full((16, 2)), _full((2,)),
        ],
        out_specs=_rows(2),
        out_shape=jax.ShapeDtypeStruct((n_pad, 2), jnp.float32),
    )


def kernel(x, edge_index, W1, b1, g1, be1, W2, b2, g2, be2, W3, b3, g3, be3,
           sW2, sb2, sW3, sb3, cW1, cb1, cW2, cb2):
    n, _ = x.shape
    e = edge_index.shape[1]
    n_pad = -(-n // 2048) * 2048
    e_pad = -(-e // (_NW * _CHUNK)) * (_NW * _CHUNK)

    pad = e_pad - e
    s_pad = jnp.concatenate([edge_index[0], jnp.zeros((pad,), jnp.int32)])
    d_pad = jnp.concatenate(
        [edge_index[1], jnp.full((pad,), n_pad - 1, jnp.int32)])
    es = jnp.stack([s_pad.reshape(-1, _CHUNK), d_pad.reshape(-1, _CHUNK)],
                   axis=1)
    x_p = jnp.pad(x, ((0, n_pad - n), (0, 0)))

    z64 = jnp.zeros((n_pad, 64), jnp.float32)
    ones16 = jnp.ones((_CHUNK, 16), jnp.float32)

    deg_p = _make_deg_kernel(n_pad, e_pad)(es, z64[:, :16], ones16)
    dinv, hp1a, hp1b = _stage_a(n_pad)(deg_p, deg_p, x_p, W1)

    agg1 = _make_agg_kernel(n_pad, e_pad, 64, 2, _C0F)(hp1a, hp1b, es, z64)
    hp2, res2 = _stage_b1(n_pad)(
        agg1, agg1, agg1, agg1, hp1a, hp1b, x_p, dinv,
        b1, g1, be1, W2, sW2, sb2)

    agg2 = _make_agg_kernel(n_pad, e_pad, 64, 1, _C0F)(hp2, es, z64)
    hp3, res3 = _stage_b2(n_pad)(
        agg2, agg2, hp2, res2, dinv, b2, g2, be2, W3, sW3, sb3)

    agg3 = _make_agg_kernel(n_pad, e_pad, 32, 1, _C0F)(hp3, es, z64[:, :32])
    out = _stage_final(n_pad)(
        agg3, agg3, hp3, res3, dinv, b3, g3, be3, cW1, cb1, cW2, cb2)

    return out[:n]


# speedup vs baseline: 1.0429x; 1.0175x over previous
import functools

import jax
import jax.numpy as jnp
from jax import lax
from jax.experimental import pallas as pl
from jax.experimental.pallas import tpu as pltpu
from jax.experimental.pallas import tpu_sc as plsc

_INV_STD = 1.0 / (1.0 + 1e-5) ** 0.5

_NW = 32
_CHUNK = 128
_NBUF = 6
_GDEPTH = 3
_RB = 512
_C0F = 0.8


def _make_deg_kernel(n_pad, e_pad):
    per_w = e_pad // _NW
    n_chunks = per_w // _CHUNK
    stripe = n_pad // 16
    mesh = plsc.VectorSubcoreMesh(core_axis_name="c", subcore_axis_name="s")

    @functools.partial(
        pl.kernel,
        mesh=mesh,
        compiler_params=pltpu.CompilerParams(use_tc_tiling_on_sc=False),
        out_type=jax.ShapeDtypeStruct((2, n_pad, 16), jnp.float32),
        scratch_types=[
            pltpu.VMEM((2, 2, _CHUNK), jnp.int32),
            pltpu.VMEM((_CHUNK, 16), jnp.float32),
            pltpu.VMEM_SHARED((n_pad, 16), jnp.float32),
            pltpu.SemaphoreType.DMA,
        ],
    )
    def deg_kernel(es_hbm, z_hbm, ones_hbm, out_hbm, sd, ones_v, acc, ssem):
        cid = lax.axis_index("c")
        sid = lax.axis_index("s")
        wid = sid * 2 + cid
        pltpu.sync_copy(ones_hbm, ones_v)
        pltpu.sync_copy(
            z_hbm.at[pl.ds(sid * stripe, stripe)],
            acc.at[pl.ds(sid * stripe, stripe)],
        )
        plsc.subcore_barrier()
        cbase = wid * n_chunks

        def copy_idx(c, slot):
            pltpu.sync_copy(es_hbm.at[cbase + c], sd.at[slot])

        def scat_start(slot):
            pltpu.async_copy(ones_v, acc.at[sd.at[slot, 1]], ssem, add=True)

        def scat_wait(slot):
            pltpu.make_async_copy(ones_v, acc.at[sd.at[slot, 1]], ssem).wait()

        copy_idx(0, 0)
        scat_start(0)
        copy_idx(1, 1)
        scat_start(1)

        def body(j, carry):
            b = j % 2
            scat_wait(b)
            copy_idx(j, b)
            scat_start(b)
            return carry

        lax.fori_loop(2, n_chunks, body, 0)
        scat_wait(0)
        scat_wait(1)
        plsc.subcore_barrier()
        pltpu.sync_copy(
            acc.at[pl.ds(sid * stripe, stripe)],
            out_hbm.at[cid, pl.ds(sid * stripe, stripe)],
        )

    return deg_kernel


def _make_agg_kernel(n_pad, e_pad, d_feat, n_slabs, c0_frac=0.5):
    per16 = e_pad // _CHUNK // 16
    cb0 = max(_NBUF, min(per16 - _NBUF, round(per16 * c0_frac)))
    cb1 = per16 - cb0
    stripe = n_pad // 16
    mesh = plsc.VectorSubcoreMesh(core_axis_name="c", subcore_axis_name="s")

    def agg_body(*refs):
        hps = refs[:n_slabs]
        es_hbm, z_hbm, out_hbm, sd, rows, acc, gsem, ssem = refs[n_slabs:]
        cid = lax.axis_index("c")
        sid = lax.axis_index("s")
        n_chunks = jnp.where(cid == 0, cb0, cb1)
        cbase = jnp.where(cid == 0, sid * cb0, 16 * cb0 + sid * cb1)

        def copy_idx(c):
            pltpu.sync_copy(es_hbm.at[cbase + c], sd.at[c % _NBUF])

        for slab in range(n_slabs):
            hp_hbm = hps[slab]

            def gather_start(c):
                pltpu.async_copy(
                    hp_hbm.at[sd.at[c % _NBUF, 0]], rows.at[c % _NBUF], gsem)

            def gather_wait(c):
                pltpu.make_async_copy(
                    hp_hbm.at[sd.at[c % _NBUF, 0]], rows.at[c % _NBUF],
                    gsem).wait()

            def scat_start(c):
                pltpu.async_copy(
                    rows.at[c % _NBUF], acc.at[sd.at[c % _NBUF, 1]], ssem,
                    add=True)

            def scat_wait(c):
                pltpu.make_async_copy(
                    rows.at[c % _NBUF], acc.at[sd.at[c % _NBUF, 1]],
                    ssem).wait()

            for c in range(_GDEPTH):
                copy_idx(c)
                gather_start(c)

            pltpu.sync_copy(
                z_hbm.at[pl.ds(sid * stripe, stripe)],
                acc.at[pl.ds(sid * stripe, stripe)],
            )
            plsc.subcore_barrier()

            def body(j, carry):
                gather_wait(j)
                scat_start(j)

                @pl.when(jnp.logical_and(j + _GDEPTH < n_chunks,
                                         j >= _GDEPTH))
                def _():
                    scat_wait(j - (_NBUF - _GDEPTH))

                @pl.when(j + _GDEPTH < n_chunks)
                def _():
                    copy_idx(j + _GDEPTH)
                    gather_start(j + _GDEPTH)

                return carry

            lax.fori_loop(0, n_chunks, body, 0)
            for c in range(_NBUF):
                scat_wait(c)
            plsc.subcore_barrier()
            pltpu.sync_copy(
                acc.at[pl.ds(sid * stripe, stripe)],
                out_hbm.at[slab, cid, pl.ds(sid * stripe, stripe)],
            )

    return functools.partial(
        pl.kernel,
        mesh=mesh,
        compiler_params=pltpu.CompilerParams(use_tc_tiling_on_sc=False),
        out_type=jax.ShapeDtypeStruct((n_slabs, 2, n_pad, d_feat),
                                      jnp.float32),
        scratch_types=[
            pltpu.VMEM((_NBUF, 2, _CHUNK), jnp.int32),
            pltpu.VMEM((_NBUF, _CHUNK, d_feat), jnp.float32),
            pltpu.VMEM_SHARED((n_pad, d_feat), jnp.float32),
            pltpu.SemaphoreType.DMA,
            pltpu.SemaphoreType.DMA,
        ],
    )(agg_body)


def _full(shape):
    nd = len(shape)
    return pl.BlockSpec(shape, lambda i, _n=nd: (0,) * _n)


def _rows(minor):
    return pl.BlockSpec((_RB, minor), lambda i: (i, 0))


def _deg_part(which):
    return pl.BlockSpec((1, _RB, 16), lambda i, _w=which: (_w, i, 0))


def _agg_part(minor, slab, which):
    return pl.BlockSpec((1, 1, _RB, minor),
                        lambda i, _s=slab, _w=which: (_s, _w, i, 0))


def _stage_a(n_pad):
    def body(d0_ref, d1_ref, x_ref, w1_ref, dinv_ref, hpa_ref, hpb_ref):
        deg = d0_ref[0] + d1_ref[0] + 1.0
        dv = lax.rsqrt(deg)
        dinv_ref[...] = dv
        xw = jnp.dot(x_ref[...], w1_ref[...], preferred_element_type=jnp.float32)
        hp = xw * dv[:, 0:1]
        hpa_ref[...] = hp[:, :64]
        hpb_ref[...] = hp[:, 64:]

    return pl.pallas_call(
        body,
        grid=(n_pad // _RB,),
        in_specs=[_deg_part(0), _deg_part(1), _rows(128), _full((128, 128))],
        out_specs=[_rows(16), _rows(64), _rows(64)],
        out_shape=[
            jax.ShapeDtypeStruct((n_pad, 16), jnp.float32),
            jax.ShapeDtypeStruct((n_pad, 64), jnp.float32),
            jax.ShapeDtypeStruct((n_pad, 64), jnp.float32),
        ],
    )


def _stage_b1(n_pad):
    def body(aa0_ref, aa1_ref, ab0_ref, ab1_ref, hpa_ref, hpb_ref, x_ref,
             dv_ref, b_ref, g_ref, be_ref, wn_ref, swn_ref, sbn_ref,
             hpn_ref, resn_ref):
        dv = dv_ref[...][:, 0:1]
        ta = aa0_ref[0, 0] + aa1_ref[0, 0] + hpa_ref[...]
        tb = ab0_ref[0, 0] + ab1_ref[0, 0] + hpb_ref[...]
        t = jnp.concatenate([ta, tb], axis=1) * dv + b_ref[...][None, :]
        t = t * (_INV_STD * g_ref[...][None, :]) + be_ref[...][None, :]
        h = jnp.maximum(t, 0.0) + x_ref[...]
        hpn_ref[...] = jnp.dot(h, wn_ref[...],
                               preferred_element_type=jnp.float32) * dv
        resn_ref[...] = jnp.dot(h, swn_ref[...],
                                preferred_element_type=jnp.float32) + sbn_ref[...][None, :]

    return pl.pallas_call(
        body,
        grid=(n_pad // _RB,),
        in_specs=[
            _agg_part(64, 0, 0), _agg_part(64, 0, 1),
            _agg_part(64, 1, 0), _agg_part(64, 1, 1),
            _rows(64), _rows(64), _rows(128),
            _rows(16), _full((128,)), _full((128,)), _full((128,)),
            _full((128, 64)), _full((128, 64)), _full((64,)),
        ],
        out_specs=[_rows(64), _rows(64)],
        out_shape=[
            jax.ShapeDtypeStruct((n_pad, 64), jnp.float32),
            jax.ShapeDtypeStruct((n_pad, 64), jnp.float32),
        ],
    )


def _stage_b2(n_pad):
    def body(a0_ref, a1_ref, hp_ref, res_ref, dv_ref, b_ref, g_ref, be_ref,
             wn_ref, swn_ref, sbn_ref, hpn_ref, resn_ref):
        dv = dv_ref[...][:, 0:1]
        t = (a0_ref[0, 0] + a1_ref[0, 0] + hp_ref[...]) * dv + b_ref[...][None, :]
        t = t * (_INV_STD * g_ref[...][None, :]) + be_ref[...][None, :]
        h = jnp.maximum(t, 0.0) + res_ref[...]
        hpn_ref[...] = jnp.dot(h, wn_ref[...],
                               preferred_element_type=jnp.float32) * dv
        resn_ref[...] = jnp.dot(h, swn_ref[...],
                                preferred_element_type=jnp.float32) + sbn_ref[...][None, :]

    return pl.pallas_call(
        body,
        grid=(n_pad // _RB,),
        in_specs=[
            _agg_part(64, 0, 0), _agg_part(64, 0, 1),
            _rows(64), _rows(64),
            _rows(16), _full((64,)), _full((64,)), _full((64,)),
            _full((64, 32)), _full((64, 32)), _full((32,)),
        ],
        out_specs=[_rows(32), _rows(32)],
        out_shape=[
            jax.ShapeDtypeStruct((n_pad, 32), jnp.float32),
            jax.ShapeDtypeStruct((n_pad, 32), jnp.float32),
        ],
    )


def _stage_final(n_pad):
    def body(a0_ref, a1_ref, hp_ref, res_ref, dv_ref, b_ref, g_ref, be_ref,
             cw1_ref, cb1_ref, cw2_ref, cb2_ref, out_ref):
        dv = dv_ref[...][:, 0:1]
        t = (a0_ref[0, 0] + a1_ref[0, 0] + hp_ref[...]) * dv + b_ref[...][None, :]
        t = t * (_INV_STD * g_ref[...][None, :]) + be_ref[...][None, :]
        h = jnp.maximum(t, 0.0) + res_ref[...]
        u = jnp.dot(h, cw1_ref[...], preferred_element_type=jnp.float32)
        u = jnp.maximum(u + cb1_ref[...][None, :], 0.0)
        v = jnp.dot(u, cw2_ref[...],
                    preferred_element_type=jnp.float32) + cb2_ref[...][None, :]
        m = jnp.max(v, axis=1, keepdims=True)
        lse = m + jnp.log(jnp.sum(jnp.exp(v - m), axis=1, keepdims=True))
        out_ref[...] = v - lse

    return pl.pallas_call(
        body,
        grid=(n_pad // _RB,),
        in_specs=[
            _agg_part(32, 0, 0), _agg_part(32, 0, 1),
            _rows(32), _rows(32),
            _rows(16), _full((32,)), _full((32,)), _full((32,)),
            _full((32, 16)), _full((16,)), _full((16, 2)), _full((2,)),
        ],
        out_specs=_rows(2),
        out_shape=jax.ShapeDtypeStruct((n_pad, 2), jnp.float32),
    )


def kernel(x, edge_index, W1, b1, g1, be1, W2, b2, g2, be2, W3, b3, g3, be3,
           sW2, sb2, sW3, sb3, cW1, cb1, cW2, cb2):
    n, _ = x.shape
    e = edge_index.shape[1]
    n_pad = -(-n // 2048) * 2048
    e_pad = -(-e // (_NW * _CHUNK)) * (_NW * _CHUNK)

    pad = e_pad - e
    s_pad = jnp.concatenate([edge_index[0], jnp.zeros((pad,), jnp.int32)])
    d_pad = jnp.concatenate(
        [edge_index[1], jnp.full((pad,), n_pad - 1, jnp.int32)])
    es = jnp.stack([s_pad.reshape(-1, _CHUNK), d_pad.reshape(-1, _CHUNK)],
                   axis=1)
    x_p = jnp.pad(x, ((0, n_pad - n), (0, 0)))

    z64 = jnp.zeros((n_pad, 64), jnp.float32)
    ones16 = jnp.ones((_CHUNK, 16), jnp.float32)

    deg_p = _make_deg_kernel(n_pad, e_pad)(es, z64[:, :16], ones16)
    dinv, hp1a, hp1b = _stage_a(n_pad)(deg_p, deg_p, x_p, W1)

    agg1 = _make_agg_kernel(n_pad, e_pad, 64, 2, _C0F)(hp1a, hp1b, es, z64)
    hp2, res2 = _stage_b1(n_pad)(
        agg1, agg1, agg1, agg1, hp1a, hp1b, x_p, dinv,
        b1, g1, be1, W2, sW2, sb2)

    agg2 = _make_agg_kernel(n_pad, e_pad, 64, 1, _C0F)(hp2, es, z64)
    hp3, res3 = _stage_b2(n_pad)(
        agg2, agg2, hp2, res2, dinv, b2, g2, be2, W3, sW3, sb3)

    agg3 = _make_agg_kernel(n_pad, e_pad, 32, 1, 2.0 / 3.0)(
        hp3, es, z64[:, :32])
    out = _stage_final(n_pad)(
        agg3, agg3, hp3, res3, dinv, b3, g3, be3, cW1, cb1, cW2, cb2)

    return out[:n]
